# baseline (speedup 1.0000x reference)
import functools

import jax
import jax.numpy as jnp
from jax import lax
from jax.experimental import pallas as pl
from jax.experimental.pallas import tpu as pltpu
from jax.experimental.pallas import tpu_sc as plsc

TOKENS = 32768
EMBED = 64
PAD = 65
EXPERTS = 4
LANES = 16
NCORES = 2
NSUB = 16
NWORK = NCORES * NSUB
TPW = TOKENS // NWORK
BLK = 256
NBLK = TPW // BLK
NCHUNK = BLK // LANES

_mesh = plsc.VectorSubcoreMesh(core_axis_name="c", subcore_axis_name="s",
                               num_cores=NCORES, num_subcores=NSUB)


def _round_bf16_even(v):
    u = plsc.bitcast(v, jnp.uint32)
    r = u + jnp.uint32(0x7FFF) + ((u >> jnp.uint32(16)) & jnp.uint32(1))
    r = r & jnp.uint32(0xFFFF0000)
    return plsc.bitcast(r, jnp.float32)


@functools.partial(
    pl.kernel,
    out_type=jax.ShapeDtypeStruct((TOKENS * 2,), jnp.int32),
    mesh=_mesh,
    scratch_types=[
        pltpu.VMEM((BLK * EMBED,), jnp.float32),
        pltpu.VMEM((BLK, PAD), jnp.float32),
        pltpu.VMEM((EXPERTS * EMBED,), jnp.float32),
        pltpu.VMEM((TPW * 2,), jnp.int32),
    ],
    compiler_params=pltpu.CompilerParams(needs_layout_passes=False),
)
def _route(x_hbm, w_hbm, out_hbm, x_v, x_p, w_v, idx_v):
    wid = lax.axis_index("s") * NCORES + lax.axis_index("c")
    base = wid * TPW
    pltpu.sync_copy(w_hbm, w_v)

    lane = lax.iota(jnp.int32, 16)
    zero_f = jnp.zeros((LANES,), jnp.float32)
    ws = []
    for e in range(EXPERTS):
        row = []
        for g in range(EMBED // LANES):
            vec = w_v[pl.ds(e * EMBED + g * LANES, LANES)]
            vec = _round_bf16_even(vec)
            row.extend(vec[j] for j in range(LANES))
        ws.append(row)

    UNROLL = 8

    def repack(i, carry):
        for u in range(UNROLL):
            t = i * UNROLL + u
            for g in range(EMBED // LANES):
                v = x_v[pl.ds(t * EMBED + g * LANES, LANES)]
                x_p[t, pl.ds(g * LANES, LANES)] = _round_bf16_even(v)
        return carry

    def chunk_of(b):
        def chunk(c, carry):
            tok = b * BLK + c * LANES + lane
            blktok = c * LANES + lane
            acc = [zero_f, zero_f, zero_f, zero_f]
            for d in range(EMBED):
                dsel = jnp.full((LANES,), d, jnp.int32)
                col = plsc.load_gather(x_p, [blktok, dsel])
                for e in range(EXPERTS):
                    acc[e] = acc[e] + col * ws[e][d]
            a0, a1, a2, a3 = acc
            m01 = jnp.maximum(a0, a1)
            i01 = jnp.where(a1 > a0, 1, 0)
            n01 = jnp.minimum(a0, a1)
            j01 = jnp.where(a1 > a0, 0, 1)
            m23 = jnp.maximum(a2, a3)
            i23 = jnp.where(a3 > a2, 3, 2)
            n23 = jnp.minimum(a2, a3)
            j23 = jnp.where(a3 > a2, 2, 3)
            cond = m23 > m01
            top1 = jnp.where(cond, i23, i01)
            sec01 = jnp.where(m23 > n01, i23, j01)
            sec23 = jnp.where(n23 > m01, j23, i01)
            top2 = jnp.where(cond, sec23, sec01)
            pos = tok * 2
            plsc.store_scatter(idx_v, [pos], top1)
            plsc.store_scatter(idx_v, [pos + 1], top2)
            return carry
        return chunk

    for b in range(NBLK):
        pltpu.sync_copy(
            x_hbm.at[pl.ds((base + b * BLK) * EMBED, BLK * EMBED)], x_v)
        lax.fori_loop(0, BLK // UNROLL, repack, 0)
        lax.fori_loop(0, NCHUNK, chunk_of(b), 0)

    pltpu.sync_copy(idx_v, out_hbm.at[pl.ds(base * 2, TPW * 2)])


def kernel(x, W):
    idx = _route(x.reshape(TOKENS * EMBED), W.reshape(EXPERTS * EMBED))
    return (jnp.zeros((TOKENS, EXPERTS), jnp.float32),
            idx.reshape(TOKENS, 2))


# speedup vs baseline: 1.9368x; 1.9368x over previous
import jax
import jax.numpy as jnp
from jax.experimental import pallas as pl

TOKENS = 32768
EMBED = 64
EXPERTS = 4
BLOCK = 2048
NSTEP = TOKENS // BLOCK


def _body(x_ref, w_ref, zero_ref, idx_ref):
    xb = x_ref[...]
    logits = jax.lax.dot_general(
        xb, w_ref[...],
        dimension_numbers=(((1,), (1,)), ((), ())),
        preferred_element_type=jnp.float32,
    )
    a0 = logits[:, 0]
    a1 = logits[:, 1]
    a2 = logits[:, 2]
    a3 = logits[:, 3]
    m01 = jnp.maximum(a0, a1)
    i01 = jnp.where(a1 > a0, 1, 0)
    n01 = jnp.minimum(a0, a1)
    j01 = jnp.where(a1 > a0, 0, 1)
    m23 = jnp.maximum(a2, a3)
    i23 = jnp.where(a3 > a2, 3, 2)
    n23 = jnp.minimum(a2, a3)
    j23 = jnp.where(a3 > a2, 2, 3)
    cond = m23 > m01
    top1 = jnp.where(cond, i23, i01)
    sec01 = jnp.where(m23 > n01, i23, j01)
    sec23 = jnp.where(n23 > m01, j23, i01)
    top2 = jnp.where(cond, sec23, sec01)
    idx_ref[...] = jnp.stack([top1, top2], axis=-1)
    zero_ref[...] = jnp.zeros((BLOCK, EXPERTS), jnp.float32)


@jax.jit
def kernel(x, W):
    zeros, idx = pl.pallas_call(
        _body,
        grid=(NSTEP,),
        in_specs=[
            pl.BlockSpec((BLOCK, EMBED), lambda i: (i, 0)),
            pl.BlockSpec((EXPERTS, EMBED), lambda i: (0, 0)),
        ],
        out_specs=[
            pl.BlockSpec((BLOCK, EXPERTS), lambda i: (i, 0)),
            pl.BlockSpec((BLOCK, 2), lambda i: (i, 0)),
        ],
        out_shape=[
            jax.ShapeDtypeStruct((TOKENS, EXPERTS), jnp.float32),
            jax.ShapeDtypeStruct((TOKENS, 2), jnp.int32),
        ],
    )(x, W)
    return zeros, idx


# speedup vs baseline: 5.8278x; 3.0089x over previous
import jax
import jax.numpy as jnp
from jax.experimental import pallas as pl

TOKENS = 32768
EMBED = 64
EXPERTS = 4
BLOCK = 2048
NSTEP = TOKENS // BLOCK


def _body(x_ref, w_ref, idx_ref):
    logits = jax.lax.dot_general(
        w_ref[...], x_ref[...],
        dimension_numbers=(((1,), (1,)), ((), ())),
        preferred_element_type=jnp.float32,
    )
    a0 = logits[0, :]
    a1 = logits[1, :]
    a2 = logits[2, :]
    a3 = logits[3, :]
    m01 = jnp.maximum(a0, a1)
    i01 = jnp.where(a1 > a0, 1, 0)
    n01 = jnp.minimum(a0, a1)
    j01 = jnp.where(a1 > a0, 0, 1)
    m23 = jnp.maximum(a2, a3)
    i23 = jnp.where(a3 > a2, 3, 2)
    n23 = jnp.minimum(a2, a3)
    j23 = jnp.where(a3 > a2, 2, 3)
    cond = m23 > m01
    top1 = jnp.where(cond, i23, i01)
    sec01 = jnp.where(m23 > n01, i23, j01)
    sec23 = jnp.where(n23 > m01, j23, i01)
    top2 = jnp.where(cond, sec23, sec01)
    idx_ref[0, :] = top1
    idx_ref[1, :] = top2


@jax.jit
def kernel(x, W):
    idx_t = pl.pallas_call(
        _body,
        grid=(NSTEP,),
        in_specs=[
            pl.BlockSpec((BLOCK, EMBED), lambda i: (i, 0)),
            pl.BlockSpec((EXPERTS, EMBED), lambda i: (0, 0)),
        ],
        out_specs=pl.BlockSpec((2, BLOCK), lambda i: (0, i)),
        out_shape=jax.ShapeDtypeStruct((2, TOKENS), jnp.int32),
    )(x, W)
    return jnp.zeros((TOKENS, EXPERTS), jnp.float32), idx_t.T


# speedup vs baseline: 12.5334x; 2.1506x over previous
import jax
import jax.numpy as jnp
from jax.experimental import pallas as pl

TOKENS = 32768
EMBED = 64
EXPERTS = 4
BLOCK = 2048
NSTEP = TOKENS // BLOCK


def _body(x_ref, w_ref, idx_ref):
    logits = jax.lax.dot_general(
        w_ref[...], x_ref[...],
        dimension_numbers=(((1,), (0,)), ((), ())),
        preferred_element_type=jnp.float32,
    )
    a0 = logits[0, :]
    a1 = logits[1, :]
    a2 = logits[2, :]
    a3 = logits[3, :]
    m01 = jnp.maximum(a0, a1)
    i01 = jnp.where(a1 > a0, 1, 0)
    n01 = jnp.minimum(a0, a1)
    j01 = jnp.where(a1 > a0, 0, 1)
    m23 = jnp.maximum(a2, a3)
    i23 = jnp.where(a3 > a2, 3, 2)
    n23 = jnp.minimum(a2, a3)
    j23 = jnp.where(a3 > a2, 2, 3)
    cond = m23 > m01
    top1 = jnp.where(cond, i23, i01)
    sec01 = jnp.where(m23 > n01, i23, j01)
    sec23 = jnp.where(n23 > m01, j23, i01)
    top2 = jnp.where(cond, sec23, sec01)
    idx_ref[0, :] = top1
    idx_ref[1, :] = top2


@jax.jit
def kernel(x, W):
    idx_t = pl.pallas_call(
        _body,
        grid=(NSTEP,),
        in_specs=[
            pl.BlockSpec((EMBED, BLOCK), lambda i: (0, i)),
            pl.BlockSpec((EXPERTS, EMBED), lambda i: (0, 0)),
        ],
        out_specs=pl.BlockSpec((2, BLOCK), lambda i: (0, i)),
        out_shape=jax.ShapeDtypeStruct((2, TOKENS), jnp.int32),
    )(x.T, W)
    return jnp.zeros((TOKENS, EXPERTS), jnp.float32), idx_t.T


# speedup vs baseline: 18.3748x; 1.4661x over previous
import jax
import jax.numpy as jnp
from jax.experimental import pallas as pl
from jax.experimental.pallas import tpu as pltpu

TOKENS = 32768
EMBED = 64
EXPERTS = 4
BLOCK = 4096
NSTEP = TOKENS // BLOCK


def _body(x_hbm, w_ref, zero_ref, idx_ref, buf, sems):
    i = pl.program_id(0)
    slot = jax.lax.rem(i, 2)

    @pl.when(i == 0)
    def _prime():
        pltpu.make_async_copy(
            x_hbm.at[:, pl.ds(0, BLOCK)], buf.at[0], sems.at[0]
        ).start()

    @pl.when(i + 1 < NSTEP)
    def _prefetch():
        pltpu.make_async_copy(
            x_hbm.at[:, pl.ds((i + 1) * BLOCK, BLOCK)],
            buf.at[1 - slot],
            sems.at[1 - slot],
        ).start()

    pltpu.make_async_copy(
        x_hbm.at[:, pl.ds(i * BLOCK, BLOCK)], buf.at[slot], sems.at[slot]
    ).wait()

    logits = jax.lax.dot_general(
        w_ref[...], buf[slot],
        dimension_numbers=(((1,), (0,)), ((), ())),
        preferred_element_type=jnp.float32,
    )
    a0 = logits[0, :]
    a1 = logits[1, :]
    a2 = logits[2, :]
    a3 = logits[3, :]
    m01 = jnp.maximum(a0, a1)
    i01 = jnp.where(a1 > a0, 1, 0)
    n01 = jnp.minimum(a0, a1)
    j01 = jnp.where(a1 > a0, 0, 1)
    m23 = jnp.maximum(a2, a3)
    i23 = jnp.where(a3 > a2, 3, 2)
    n23 = jnp.minimum(a2, a3)
    j23 = jnp.where(a3 > a2, 2, 3)
    cond = m23 > m01
    top1 = jnp.where(cond, i23, i01)
    sec01 = jnp.where(m23 > n01, i23, j01)
    sec23 = jnp.where(n23 > m01, j23, i01)
    top2 = jnp.where(cond, sec23, sec01)
    idx_ref[0, :] = top1
    idx_ref[1, :] = top2
    zero_ref[...] = jnp.zeros((EXPERTS, BLOCK), jnp.float32)


@jax.jit
def kernel(x, W):
    zeros_t, idx_t = pl.pallas_call(
        _body,
        grid=(NSTEP,),
        in_specs=[
            pl.BlockSpec(memory_space=pltpu.MemorySpace.HBM),
            pl.BlockSpec((EXPERTS, EMBED), lambda i: (0, 0)),
        ],
        out_specs=[
            pl.BlockSpec((EXPERTS, BLOCK), lambda i: (0, i)),
            pl.BlockSpec((2, BLOCK), lambda i: (0, i)),
        ],
        out_shape=[
            jax.ShapeDtypeStruct((EXPERTS, TOKENS), jnp.float32),
            jax.ShapeDtypeStruct((2, TOKENS), jnp.int32),
        ],
        scratch_shapes=[
            pltpu.VMEM((2, EMBED, BLOCK), jnp.float32),
            pltpu.SemaphoreType.DMA((2,)),
        ],
    )(pltpu.with_memory_space_constraint(x.T, pltpu.MemorySpace.HBM), W)
    return zeros_t.T, idx_t.T


# speedup vs baseline: 24.2889x; 1.3219x over previous
import jax
import jax.numpy as jnp
from jax.experimental import pallas as pl
from jax.experimental.pallas import tpu as pltpu

TOKENS = 32768
EMBED = 64
EXPERTS = 4
BLOCK = 8192
NSTEP = TOKENS // BLOCK


def _body(x_hbm, w_ref, zero_ref, idx_ref, buf, sems):
    i = pl.program_id(0)
    slot = jax.lax.rem(i, 2)

    @pl.when(i == 0)
    def _prime():
        pltpu.make_async_copy(
            x_hbm.at[:, pl.ds(0, BLOCK)], buf.at[0], sems.at[0]
        ).start()

    @pl.when(i + 1 < NSTEP)
    def _prefetch():
        pltpu.make_async_copy(
            x_hbm.at[:, pl.ds((i + 1) * BLOCK, BLOCK)],
            buf.at[1 - slot],
            sems.at[1 - slot],
        ).start()

    pltpu.make_async_copy(
        x_hbm.at[:, pl.ds(i * BLOCK, BLOCK)], buf.at[slot], sems.at[slot]
    ).wait()

    logits = jax.lax.dot_general(
        w_ref[...], buf[slot],
        dimension_numbers=(((1,), (0,)), ((), ())),
        preferred_element_type=jnp.float32,
    )
    a0 = logits[0, :]
    a1 = logits[1, :]
    a2 = logits[2, :]
    a3 = logits[3, :]
    m01 = jnp.maximum(a0, a1)
    i01 = jnp.where(a1 > a0, 1, 0)
    n01 = jnp.minimum(a0, a1)
    j01 = jnp.where(a1 > a0, 0, 1)
    m23 = jnp.maximum(a2, a3)
    i23 = jnp.where(a3 > a2, 3, 2)
    n23 = jnp.minimum(a2, a3)
    j23 = jnp.where(a3 > a2, 2, 3)
    cond = m23 > m01
    top1 = jnp.where(cond, i23, i01)
    sec01 = jnp.where(m23 > n01, i23, j01)
    sec23 = jnp.where(n23 > m01, j23, i01)
    top2 = jnp.where(cond, sec23, sec01)
    idx_ref[0, :] = top1
    idx_ref[1, :] = top2
    zero_ref[...] = jnp.zeros((EXPERTS, BLOCK), jnp.float32)


@jax.jit
def kernel(x, W):
    zeros_t, idx_t = pl.pallas_call(
        _body,
        grid=(NSTEP,),
        in_specs=[
            pl.BlockSpec(memory_space=pltpu.MemorySpace.HBM),
            pl.BlockSpec((EXPERTS, EMBED), lambda i: (0, 0)),
        ],
        out_specs=[
            pl.BlockSpec((EXPERTS, BLOCK), lambda i: (0, i)),
            pl.BlockSpec((2, BLOCK), lambda i: (0, i)),
        ],
        out_shape=[
            jax.ShapeDtypeStruct((EXPERTS, TOKENS), jnp.float32),
            jax.ShapeDtypeStruct((2, TOKENS), jnp.int32),
        ],
        scratch_shapes=[
            pltpu.VMEM((2, EMBED, BLOCK), jnp.float32),
            pltpu.SemaphoreType.DMA((2,)),
        ],
    )(pltpu.with_memory_space_constraint(x.T, pltpu.MemorySpace.HBM), W)
    return zeros_t.T, idx_t.T


# speedup vs baseline: 26.7175x; 1.1000x over previous
import jax
import jax.numpy as jnp
from jax.experimental import pallas as pl
from jax.experimental.pallas import tpu as pltpu

TOKENS = 32768
EMBED = 64
EXPERTS = 4
BLOCK = 16384
NSTEP = TOKENS // BLOCK


def _body(x_hbm, w_ref, zero_ref, idx_ref, buf, sems):
    i = pl.program_id(0)
    slot = jax.lax.rem(i, 2)

    @pl.when(i == 0)
    def _prime():
        pltpu.make_async_copy(
            x_hbm.at[:, pl.ds(0, BLOCK)], buf.at[0], sems.at[0]
        ).start()

    @pl.when(i + 1 < NSTEP)
    def _prefetch():
        pltpu.make_async_copy(
            x_hbm.at[:, pl.ds((i + 1) * BLOCK, BLOCK)],
            buf.at[1 - slot],
            sems.at[1 - slot],
        ).start()

    pltpu.make_async_copy(
        x_hbm.at[:, pl.ds(i * BLOCK, BLOCK)], buf.at[slot], sems.at[slot]
    ).wait()

    logits = jax.lax.dot_general(
        w_ref[...], buf[slot],
        dimension_numbers=(((1,), (0,)), ((), ())),
        preferred_element_type=jnp.float32,
    )
    a0 = logits[0, :]
    a1 = logits[1, :]
    a2 = logits[2, :]
    a3 = logits[3, :]
    m01 = jnp.maximum(a0, a1)
    i01 = jnp.where(a1 > a0, 1, 0)
    n01 = jnp.minimum(a0, a1)
    j01 = jnp.where(a1 > a0, 0, 1)
    m23 = jnp.maximum(a2, a3)
    i23 = jnp.where(a3 > a2, 3, 2)
    n23 = jnp.minimum(a2, a3)
    j23 = jnp.where(a3 > a2, 2, 3)
    cond = m23 > m01
    top1 = jnp.where(cond, i23, i01)
    sec01 = jnp.where(m23 > n01, i23, j01)
    sec23 = jnp.where(n23 > m01, j23, i01)
    top2 = jnp.where(cond, sec23, sec01)
    idx_ref[0, :] = top1
    idx_ref[1, :] = top2
    zero_ref[...] = jnp.zeros((EXPERTS, BLOCK), jnp.float32)


@jax.jit
def kernel(x, W):
    zeros_t, idx_t = pl.pallas_call(
        _body,
        grid=(NSTEP,),
        in_specs=[
            pl.BlockSpec(memory_space=pltpu.MemorySpace.HBM),
            pl.BlockSpec((EXPERTS, EMBED), lambda i: (0, 0)),
        ],
        out_specs=[
            pl.BlockSpec((EXPERTS, BLOCK), lambda i: (0, i)),
            pl.BlockSpec((2, BLOCK), lambda i: (0, i)),
        ],
        out_shape=[
            jax.ShapeDtypeStruct((EXPERTS, TOKENS), jnp.float32),
            jax.ShapeDtypeStruct((2, TOKENS), jnp.int32),
        ],
        scratch_shapes=[
            pltpu.VMEM((2, EMBED, BLOCK), jnp.float32),
            pltpu.SemaphoreType.DMA((2,)),
        ],
    )(pltpu.with_memory_space_constraint(x.T, pltpu.MemorySpace.HBM), W)
    return zeros_t.T, idx_t.T
